# Initial kernel scaffold; baseline (speedup 1.0000x reference)
import jax
import jax.numpy as jnp
from jax.experimental import pallas as pl


def kernel(z, pos, batch, params):
    raise NotImplementedError("write your pallas kernel here")


# speedup vs baseline: 2.6527x; 2.6527x over previous
import functools
import math

import jax
import jax.numpy as jnp
from jax import lax
from jax.experimental import pallas as pl
from jax.experimental.pallas import tpu as pltpu
from jax.experimental.pallas import tpu_sc as plsc

N = 2000
H = 600
NG = 50
NI = 6
CUTOFF = 10.0
K = 64
E = N * K
HP = 640
LN2 = math.log(2.0)
SPACING = CUTOFF / (NG - 1)
COEFF = -0.5 / SPACING**2

_pallas_call = pl.pallas_call

T = 40
BE = T * K
GRID = E // BE


def _ssp(x):
    return jnp.maximum(x, 0.0) + jnp.log1p(jnp.exp(-jnp.abs(x))) - LN2


def _dot(a, b):
    return lax.dot_general(a, b, (((1,), (0,)), ((), ())),
                           preferred_element_type=jnp.float32)


def _mm_body(h_ref, w_ref, o_ref):
    o_ref[...] = _dot(h_ref[...], w_ref[...])


def _matmul(h, w):
    return _pallas_call(
        _mm_body,
        out_shape=jax.ShapeDtypeStruct((h.shape[0], w.shape[1]), jnp.float32),
    )(h, w)


def _gather(xl, src):
    info = plsc.get_sparse_core_info()
    nw = info.num_cores * info.num_subcores
    per_w = E // nw
    ch = 80
    mesh = plsc.VectorSubcoreMesh(core_axis_name="c", subcore_axis_name="s")

    @functools.partial(
        pl.kernel,
        out_type=jax.ShapeDtypeStruct((E, HP), jnp.float32),
        mesh=mesh,
        scratch_types=[
            pltpu.VMEM((ch,), jnp.int32),
            pltpu.VMEM((ch, HP), jnp.float32),
            pltpu.SemaphoreType.DMA,
        ],
    )
    def k(x_hbm, i_hbm, o_hbm, idx_v, rows_v, sem):
        wid = lax.axis_index("s") * info.num_cores + lax.axis_index("c")
        base = wid * per_w

        def step(j, carry):
            off = base + j * ch
            pltpu.sync_copy(i_hbm.at[pl.ds(off, ch)], idx_v)
            pltpu.async_copy(x_hbm.at[idx_v], rows_v, sem).wait()
            pltpu.sync_copy(rows_v, o_hbm.at[pl.ds(off, ch)])
            return carry

        lax.fori_loop(0, per_w // ch, step, 0)

    return k(xl, src)


def _cfconv_body(d_ref, vm_ref, g_ref, w1_ref, b1_ref, w2_ref, b2_ref, o_ref):
    d = d_ref[...]
    off = lax.broadcasted_iota(jnp.int32, (1, NG), 1).astype(jnp.float32) * SPACING
    ea = jnp.exp(COEFF * (d - off) ** 2)
    w = _ssp(_dot(ea, w1_ref[...]) + b1_ref[...])
    w = _dot(w, w2_ref[...]) + b2_ref[...]
    cv = 0.5 * (jnp.cos(d * (math.pi / CUTOFF)) + 1.0) * vm_ref[...]
    msg = g_ref[:, :H] * (w * cv)
    o_ref[...] = jnp.sum(msg.reshape(T, K, H), axis=1)


def _cfconv(d_e, vm_e, g, w1, b1, w2, b2):
    return _pallas_call(
        _cfconv_body,
        grid=(GRID,),
        in_specs=[
            pl.BlockSpec((BE, 1), lambda i: (i, 0)),
            pl.BlockSpec((BE, 1), lambda i: (i, 0)),
            pl.BlockSpec((BE, HP), lambda i: (i, 0)),
            pl.BlockSpec((NG, H), lambda i: (0, 0)),
            pl.BlockSpec((1, H), lambda i: (0, 0)),
            pl.BlockSpec((H, H), lambda i: (0, 0)),
            pl.BlockSpec((1, H), lambda i: (0, 0)),
        ],
        out_specs=pl.BlockSpec((T, H), lambda i: (i, 0)),
        out_shape=jax.ShapeDtypeStruct((N, H), jnp.float32),
    )(d_e, vm_e, g, w1, b1, w2, b2)


def _update_body(agg_ref, h_ref, l2w_ref, l2b_ref, lw_ref, lb_ref, o_ref):
    t = _ssp(_dot(agg_ref[...], l2w_ref[...]) + l2b_ref[...])
    o_ref[...] = h_ref[...] + _dot(t, lw_ref[...]) + lb_ref[...]


def _update(agg, h, l2w, l2b, lw, lb):
    return _pallas_call(
        _update_body,
        out_shape=jax.ShapeDtypeStruct((N, H), jnp.float32),
    )(agg, h, l2w, l2b, lw, lb)


def kernel(z, pos, batch, params):
    diff = pos[:, None, :] - pos[None, :, :]
    d2 = jnp.sum(diff * diff, axis=-1)
    dm = jnp.sqrt(jnp.maximum(d2, 1e-12))
    invalid = jnp.eye(N, dtype=bool) | (batch[:, None] != batch[None, :])
    dm = jnp.where(invalid, 1e9, dm)
    negd, idx = lax.top_k(-dm, K)
    d = -negd
    vm = (d < CUTOFF).astype(jnp.float32)
    src = idx.reshape(-1).astype(jnp.int32)
    d_e = d.reshape(E, 1)
    vm_e = vm.reshape(E, 1)

    p = params
    h = p['emb'][z]
    for i in range(NI):
        w1pad = jnp.pad(p['lin1_w'][i], ((0, 0), (0, HP - H)))
        xl = _matmul(h, w1pad)
        g = _gather(xl, src)
        agg = _cfconv(d_e, vm_e, g,
                      p['mlp_w1'][i], p['mlp_b1'][i].reshape(1, H),
                      p['mlp_w2'][i], p['mlp_b2'][i].reshape(1, H))
        h = _update(agg, h,
                    p['lin2_w'][i], p['lin2_b'][i].reshape(1, H),
                    p['lin_w'][i], p['lin_b'][i].reshape(1, H))
    return h
